# baseline (speedup 1.0000x reference)
import functools

import jax
import jax.numpy as jnp
from jax import lax
from jax.experimental import pallas as pl
from jax.experimental.pallas import tpu as pltpu
from jax.experimental.pallas import tpu_sc as plsc

B = 32
SEQ = 512
PRED = 192
D = 32
H = 4
CD = 32
K = 512
COMMIT_W = 1.0
ORTHO_W = 0.8

TOK = B * H * SEQ
_SC = plsc.get_sparse_core_info()
_NW = _SC.num_cores * _SC.num_subcores
_BPW = TOK // _NW


def _stage1_kernel(x_ref, w_in2_ref, b_in2_ref, cb_ref,
                   idx_ref, aux_ref, cnorm_ref):
    i = pl.program_id(0)

    @pl.when(i == 0)
    def _prep():
        for h in range(H):
            cbh = cb_ref[h]
            cnorm_ref[h, :] = jnp.sum(cbh * cbh, axis=1)

    xb = x_ref[0]
    last = xb[SEQ - 1:SEQ, :]
    x0 = xb - last
    xps = (jnp.dot(x0, w_in2_ref[...],
                   preferred_element_type=jnp.float32)
           + b_in2_ref[...])

    commit = jnp.sum(xps * xps) * 0.25
    iota = jax.lax.broadcasted_iota(jnp.int32, (SEQ, K), 1)
    for h in range(H):
        xhs = xps[:, h * CD:(h + 1) * CD]
        cb = cb_ref[h]
        dots2 = jnp.dot(xhs, cb.T,
                        preferred_element_type=jnp.float32)
        dist2 = dots2 - cnorm_ref[h:h + 1, :]
        m = jnp.max(dist2, axis=1, keepdims=True)
        idxh = jnp.min(jnp.where(dist2 == m, iota, K),
                       axis=1, keepdims=True)
        idx_ref[0, h] = idxh + h * K
        commit = commit - jnp.sum(m)

    lane = jax.lax.broadcasted_iota(jnp.int32, (128,), 0)
    aux_ref[0, 0, :] = jnp.where(lane == 0, commit, 0.0)

    @pl.when(i < H)
    def _ortho():
        cb = cb_ref[i]
        norm = jnp.sqrt(jnp.sum(cb * cb, axis=1, keepdims=True))
        normed = cb / norm
        cos = jnp.dot(normed, normed.T,
                      preferred_element_type=jnp.float32)
        osum = jnp.sum(cos * cos)
        aux_ref[0, 0, :] = (jnp.where(lane == 0, commit, 0.0)
                            + jnp.where(lane == 1, osum, 0.0))


@functools.partial(
    pl.kernel,
    mesh=plsc.VectorSubcoreMesh(core_axis_name="c", subcore_axis_name="s"),
    out_type=jax.ShapeDtypeStruct((TOK, 128), jnp.float32),
    scratch_types=[
        pltpu.VMEM((_BPW,), jnp.int32),
        pltpu.VMEM((512, 128), jnp.float32),
        pltpu.SemaphoreType.DMA,
    ],
)
def _sc_gather(table_hbm, idx_hbm, out_hbm, idx_v, rows_v, sem):
    wid = lax.axis_index("s") * _SC.num_cores + lax.axis_index("c")
    base = wid * _BPW
    pltpu.sync_copy(idx_hbm.at[pl.ds(base, _BPW)], idx_v)
    for chunk in range(_BPW // 512):
        pltpu.async_copy(
            table_hbm.at[idx_v.at[pl.ds(chunk * 512, 512)]],
            rows_v, sem).wait()
        pltpu.sync_copy(rows_v,
                        out_hbm.at[pl.ds(base + chunk * 512, 512)])


def _stage3_kernel(quant_ref, xlast_ref, w_out_ref, b_out_ref,
                   w_lin_ref, b_lin_ref, out_ref):
    qh = quant_ref[0]
    q = jnp.concatenate([qh[h][:, :CD] for h in range(H)],
                        axis=1)
    qo = (jnp.dot(q, w_out_ref[...],
                  preferred_element_type=jnp.float32)
          + b_out_ref[...])
    y = jnp.dot(w_lin_ref[...], qo,
                preferred_element_type=jnp.float32)
    out_ref[0] = y + b_lin_ref[...] + xlast_ref[0]


@jax.jit
def kernel(x, W_in, b_in, W_out, b_out, codebook, W_lin, b_lin):
    idx, aux = pl.pallas_call(
        _stage1_kernel,
        grid=(B,),
        in_specs=[
            pl.BlockSpec((1, SEQ, D), lambda i: (i, 0, 0)),
            pl.BlockSpec((D, H * CD), lambda i: (0, 0)),
            pl.BlockSpec((1, H * CD), lambda i: (0, 0)),
            pl.BlockSpec((H, K, CD), lambda i: (0, 0, 0)),
        ],
        out_specs=[
            pl.BlockSpec((1, H, SEQ, 1), lambda i: (i, 0, 0, 0)),
            pl.BlockSpec((1, 1, 128), lambda i: (i, 0, 0)),
        ],
        out_shape=[
            jax.ShapeDtypeStruct((B, H, SEQ, 1), jnp.int32),
            jax.ShapeDtypeStruct((B, 1, 128), jnp.float32),
        ],
        scratch_shapes=[pltpu.VMEM((H, K), jnp.float32)],
    )(x, W_in + W_in, (b_in + b_in).reshape(1, H * CD), codebook)

    table = jnp.pad(codebook.reshape(H * K, CD), ((0, 0), (0, 128 - CD)))
    quant = _sc_gather(table, idx.reshape(TOK))

    out = pl.pallas_call(
        _stage3_kernel,
        grid=(B,),
        in_specs=[
            pl.BlockSpec((1, H, SEQ, 128), lambda i: (i, 0, 0, 0)),
            pl.BlockSpec((1, 1, D), lambda i: (i, 0, 0)),
            pl.BlockSpec((H * CD, D), lambda i: (0, 0)),
            pl.BlockSpec((1, D), lambda i: (0, 0)),
            pl.BlockSpec((PRED, SEQ), lambda i: (0, 0)),
            pl.BlockSpec((PRED, 1), lambda i: (0, 0)),
        ],
        out_specs=pl.BlockSpec((1, PRED, D), lambda i: (i, 0, 0)),
        out_shape=jax.ShapeDtypeStruct((B, PRED, D), jnp.float32),
    )(quant.reshape(B, H, SEQ, 128), x[:, -1:, :],
      W_out, b_out.reshape(1, D), W_lin, b_lin.reshape(PRED, 1))

    commit = jnp.sum(aux[:, 0, 0]) / (B * H * SEQ * CD)
    ortho = jnp.sum(aux[:H, 0, 1]) / (H * K * K) - 1.0 / K
    loss = COMMIT_W * commit + ORTHO_W * ortho
    return out, loss


# speedup vs baseline: 2.3221x; 2.3221x over previous
import jax
import jax.numpy as jnp
from jax.experimental import pallas as pl
from jax.experimental.pallas import tpu as pltpu

B = 32
SEQ = 512
PRED = 192
D = 32
H = 4
CD = 32
K = 512
COMMIT_W = 1.0
ORTHO_W = 0.8


def _fused_kernel(x_ref, w_in2_ref, b_in2_ref, w_out_ref, b_out_ref,
                  cb_ref, w_lin_ref, b_lin_ref,
                  out_ref, aux_ref, cnorm_ref):
    i = pl.program_id(0)

    @pl.when(i == 0)
    def _prep():
        for h in range(H):
            cbh = cb_ref[h]
            cnorm_ref[h, :] = jnp.sum(cbh * cbh, axis=1)

    xb = x_ref[0]
    last = xb[SEQ - 1:SEQ, :]
    x0 = xb - last
    xps = (jnp.dot(x0, w_in2_ref[...],
                   preferred_element_type=jnp.float32)
           + b_in2_ref[...])

    commit = jnp.sum(xps * xps) * 0.25
    quants = []
    for h in range(H):
        xhs = xps[:, h * CD:(h + 1) * CD]
        cb = cb_ref[h]
        dots2 = jnp.dot(xhs, cb.T,
                        preferred_element_type=jnp.float32)
        dist2 = dots2 - cnorm_ref[h:h + 1, :]
        m = jnp.max(dist2, axis=1, keepdims=True)
        onehot = (dist2 == m).astype(jnp.float32)
        quant = jnp.dot(onehot, cb,
                        preferred_element_type=jnp.float32)
        commit = commit - jnp.sum(m)
        quants.append(quant)

    q = jnp.concatenate(quants, axis=1)
    qo = (jnp.dot(q, w_out_ref[...],
                  preferred_element_type=jnp.float32)
          + b_out_ref[...])
    y = jnp.dot(w_lin_ref[...], qo,
                preferred_element_type=jnp.float32)
    out_ref[0] = y + b_lin_ref[...] + last

    lane = jax.lax.broadcasted_iota(jnp.int32, (128,), 0)
    aux_ref[0, 0, :] = jnp.where(lane == 0, commit, 0.0)

    @pl.when(i < H)
    def _ortho():
        cb = cb_ref[i]
        norm = jnp.sqrt(jnp.sum(cb * cb, axis=1, keepdims=True))
        normed = cb / norm
        cos = jnp.dot(normed, normed.T,
                      preferred_element_type=jnp.float32)
        osum = jnp.sum(cos * cos)
        aux_ref[0, 0, :] = (jnp.where(lane == 0, commit, 0.0)
                            + jnp.where(lane == 1, osum, 0.0))


@jax.jit
def kernel(x, W_in, b_in, W_out, b_out, codebook, W_lin, b_lin):
    out, aux = pl.pallas_call(
        _fused_kernel,
        grid=(B,),
        in_specs=[
            pl.BlockSpec((1, SEQ, D), lambda i: (i, 0, 0)),
            pl.BlockSpec((D, H * CD), lambda i: (0, 0)),
            pl.BlockSpec((1, H * CD), lambda i: (0, 0)),
            pl.BlockSpec((H * CD, D), lambda i: (0, 0)),
            pl.BlockSpec((1, D), lambda i: (0, 0)),
            pl.BlockSpec((H, K, CD), lambda i: (0, 0, 0)),
            pl.BlockSpec((PRED, SEQ), lambda i: (0, 0)),
            pl.BlockSpec((PRED, 1), lambda i: (0, 0)),
        ],
        out_specs=[
            pl.BlockSpec((1, PRED, D), lambda i: (i, 0, 0)),
            pl.BlockSpec((1, 1, 128), lambda i: (i, 0, 0)),
        ],
        out_shape=[
            jax.ShapeDtypeStruct((B, PRED, D), jnp.float32),
            jax.ShapeDtypeStruct((B, 1, 128), jnp.float32),
        ],
        scratch_shapes=[pltpu.VMEM((H, K), jnp.float32)],
    )(x, W_in + W_in, (b_in + b_in).reshape(1, H * CD),
      W_out, b_out.reshape(1, D),
      codebook, W_lin, b_lin.reshape(PRED, 1))

    commit = jnp.sum(aux[:, 0, 0]) / (B * H * SEQ * CD)
    ortho = jnp.sum(aux[:H, 0, 1]) / (H * K * K) - 1.0 / K
    loss = COMMIT_W * commit + ORTHO_W * ortho
    return out, loss
